# baseline (speedup 1.0000x reference)
import functools

import jax
import jax.numpy as jnp
from jax import lax
from jax.experimental import pallas as pl
from jax.experimental.pallas import tpu as pltpu
from jax.experimental.pallas import tpu_sc as plsc

N = 10000
E = 320000
NTILES = 32
KROWS = 80
EPAD = NTILES * KROWS * 128
K0 = 104
K1 = 2 * KROWS - K0
KMAX = max(K0, K1)
GR = 4
assert K0 % (2 * GR) == 0 and K1 % (2 * GR) == 0
TOTROW = 16 * (K0 + K1)
FLATROW = max(TOTROW, 16 * K0 + 15 * K1 + KMAX)
FLATROW += (-FLATROW) % GR
NROWS = 10240
RPT = NROWS // 16
D1 = 48
D2 = 32
DD = 8

@functools.cache
def _get_mesh():
    return plsc.VectorSubcoreMesh(core_axis_name="c", subcore_axis_name="s")


@functools.cache
def _make_sc_agg(D):
    @functools.partial(
        pl.kernel,
        out_type=jax.ShapeDtypeStruct((2, NROWS, D), jnp.float32),
        mesh=_get_mesh(),
        compiler_params=pltpu.CompilerParams(use_tc_tiling_on_sc=False),
        scratch_types=[
            pltpu.VMEM((KMAX // GR, 1, GR * 128), jnp.int32),
            pltpu.VMEM((KMAX // GR, 1, GR * 128), jnp.int32),
            pltpu.VMEM((1, GR * 128, D), jnp.float32),
            pltpu.VMEM((1, GR * 128, D), jnp.float32),
            pltpu.VMEM_SHARED((1, NROWS, D), jnp.float32),
            pltpu.SemaphoreType.DMA,
            pltpu.SemaphoreType.DMA,
        ],
    )
    def agg(u_hbm, src_hbm, dst_hbm, zero_hbm, out_hbm, srcv, dstv,
            bufa, bufb, acc, sema, semb):
        c = lax.axis_index("c")
        s = lax.axis_index("s")
        start = jnp.where(c == 0, s * (K0 // GR), (16 * K0 + s * K1) // GR)
        ngrp = jnp.where(c == 0, K0 // GR, K1 // GR)

        pltpu.sync_copy(src_hbm.at[pl.ds(start, KMAX // GR)], srcv)
        pltpu.sync_copy(dst_hbm.at[pl.ds(start, KMAX // GR)], dstv)

        pltpu.sync_copy(zero_hbm, acc.at[0, pl.ds(s * RPT, RPT)])

        plsc.subcore_barrier()

        pltpu.async_copy(u_hbm.at[srcv.at[0]], bufa, sema)
        pltpu.async_copy(u_hbm.at[srcv.at[1]], bufb, semb)

        def _pair(p, carry):
            g = p * 2
            pltpu.make_async_copy(u_hbm.at[srcv.at[g]], bufa, sema).wait()
            pltpu.sync_copy(bufa, acc.at[dstv.at[g]], add=True)

            @pl.when(g + 2 < ngrp)
            def _():
                pltpu.async_copy(u_hbm.at[srcv.at[g + 2]], bufa, sema)

            pltpu.make_async_copy(u_hbm.at[srcv.at[g + 1]], bufb, semb).wait()
            pltpu.sync_copy(bufb, acc.at[dstv.at[g + 1]], add=True)

            @pl.when(g + 3 < ngrp)
            def _():
                pltpu.async_copy(u_hbm.at[srcv.at[g + 3]], bufb, semb)

            return carry

        lax.fori_loop(0, ngrp // 2, _pair, 0)

        plsc.subcore_barrier()

        pltpu.sync_copy(acc.at[0, pl.ds(s * RPT, RPT)],
                        out_hbm.at[c, pl.ds(s * RPT, RPT)])

    return agg


@functools.cache
def _make_sc_deg():

    @functools.partial(
        pl.kernel,
        out_type=jax.ShapeDtypeStruct((2, NROWS, DD), jnp.float32),
        mesh=_get_mesh(),
        compiler_params=pltpu.CompilerParams(use_tc_tiling_on_sc=False),
        scratch_types=[
            pltpu.VMEM((KROWS, 128), jnp.int32),
            pltpu.VMEM((128, DD), jnp.float32),
            pltpu.VMEM_SHARED((NROWS, DD), jnp.float32),
        ],
    )
    def deg(dst_hbm, ones_hbm, zero_hbm, out_hbm, dstv, ones, acc):
        c = lax.axis_index("c")
        s = lax.axis_index("s")
        wid = s * 2 + c

        pltpu.sync_copy(dst_hbm.at[wid], dstv)
        pltpu.sync_copy(ones_hbm, ones)
        pltpu.sync_copy(zero_hbm, acc.at[pl.ds(s * RPT, RPT)])

        plsc.subcore_barrier()

        @pl.loop(0, KROWS)
        def _(j):
            pltpu.sync_copy(ones, acc.at[dstv.at[j]], add=True)

        plsc.subcore_barrier()

        pltpu.sync_copy(acc.at[pl.ds(s * RPT, RPT)],
                        out_hbm.at[c, pl.ds(s * RPT, RPT)])

    return deg


def _scale_body(degp_ref, x_ref, w_ref, u_ref, dinv_ref):
    deg = degp_ref[0, :N, 0:1] + degp_ref[1, :N, 0:1] + 1.0
    dinv = lax.rsqrt(deg)
    dinv_ref[...] = dinv
    xw = jnp.dot(x_ref[...], w_ref[...], preferred_element_type=jnp.float32)
    u_ref[...] = xw * dinv


def _tc_scale(degp, x, w1p):
    return pl.pallas_call(
        _scale_body,
        out_shape=(jax.ShapeDtypeStruct((N, D1), jnp.float32),
                   jax.ShapeDtypeStruct((N, 1), jnp.float32)),
    )(degp, x, w1p)


def _layer_body(aggp_ref, u_ref, dinv_ref, b1_ref, w2_ref, u2_ref):
    dinv = dinv_ref[...]
    a = aggp_ref[0, :N, :] + aggp_ref[1, :N, :] + u_ref[...]
    h = jnp.maximum(a * dinv + b1_ref[...], 0.0)
    u2_ref[...] = jnp.dot(h, w2_ref[...],
                          preferred_element_type=jnp.float32) * dinv


def _tc_layer(aggp, u1, dinv, b1p, w2p):
    return pl.pallas_call(
        _layer_body,
        out_shape=jax.ShapeDtypeStruct((N, D2), jnp.float32),
    )(aggp, u1, dinv, b1p, w2p)


def _final_body(aggp_ref, u2_ref, dinv_ref, b2_ref, o_ref):
    a = aggp_ref[0, :N, :] + aggp_ref[1, :N, :] + u2_ref[...]
    o_ref[...] = (a * dinv_ref[...] + b2_ref[...])[:, :20]


def _tc_final(aggp, u2, dinv, b2p):
    return pl.pallas_call(
        _final_body,
        out_shape=jax.ShapeDtypeStruct((N, 20), jnp.float32),
    )(aggp, u2, dinv, b2p)


def kernel(x, edge_index, W1, b1, W2, b2):
    src = edge_index[0].astype(jnp.int32)
    dst = edge_index[1].astype(jnp.int32)
    npad = EPAD - E
    srcf = jnp.concatenate([src, jnp.zeros((npad,), jnp.int32)])
    dstf = jnp.concatenate(
        [dst, N + (jnp.arange(npad, dtype=jnp.int32) % (NROWS - N))])
    extra = (FLATROW - TOTROW) * 128
    srcp = jnp.concatenate(
        [srcf, jnp.zeros((extra,), jnp.int32)]
    ).reshape(FLATROW // GR, 1, GR * 128)
    dstp = jnp.concatenate(
        [dstf, jnp.full((extra,), N, jnp.int32)]
    ).reshape(FLATROW // GR, 1, GR * 128)
    dstp_deg = dstf.reshape(NTILES, KROWS, 128)

    w1p = jnp.pad(W1, ((0, 0), (0, D1 - 40)))
    b1p = jnp.pad(b1, (0, D1 - 40)).reshape(1, D1)
    w2p = jnp.pad(W2, ((0, D1 - 40), (0, D2 - 20)))
    b2p = jnp.pad(b2, (0, D2 - 20)).reshape(1, D2)

    ones_c = jnp.ones((128, DD), jnp.float32)
    zero_c = jnp.zeros((RPT, DD), jnp.float32)
    zero1 = jnp.zeros((RPT, D1), jnp.float32)
    zero2 = jnp.zeros((RPT, D2), jnp.float32)
    degp = _make_sc_deg()(dstp_deg, ones_c, zero_c)
    u1, dinv = _tc_scale(degp, x, w1p)
    agg1 = _make_sc_agg(D1)(u1.reshape(1, N, D1), srcp, dstp, zero1)
    u2 = _tc_layer(agg1, u1, dinv, b1p, w2p)
    agg2 = _make_sc_agg(D2)(u2.reshape(1, N, D2), srcp, dstp, zero2)
    return _tc_final(agg2, u2, dinv, b2p)


# speedup vs baseline: 1.0289x; 1.0289x over previous
import functools

import jax
import jax.numpy as jnp
from jax import lax
from jax.experimental import pallas as pl
from jax.experimental.pallas import tpu as pltpu
from jax.experimental.pallas import tpu_sc as plsc

N = 10000
E = 320000
NTILES = 32
KROWS = 80
EPAD = NTILES * KROWS * 128
K0 = 104
K1 = 2 * KROWS - K0
KMAX = max(K0, K1)
GR = 4
assert K0 % (2 * GR) == 0 and K1 % (2 * GR) == 0
TOTROW = 16 * (K0 + K1)
FLATROW = max(TOTROW, 16 * K0 + 15 * K1 + KMAX)
FLATROW += (-FLATROW) % GR
NROWS = 10240
RPT = NROWS // 16
D1 = 48
D2 = 32
DD = 8

@functools.cache
def _get_mesh():
    return plsc.VectorSubcoreMesh(core_axis_name="c", subcore_axis_name="s")


@functools.cache
def _make_sc_agg(D):
    @functools.partial(
        pl.kernel,
        out_type=jax.ShapeDtypeStruct((2, NROWS, D), jnp.float32),
        mesh=_get_mesh(),
        compiler_params=pltpu.CompilerParams(use_tc_tiling_on_sc=False),
        scratch_types=[
            pltpu.VMEM((KMAX // GR, 1, GR * 128), jnp.int32),
            pltpu.VMEM((KMAX // GR, 1, GR * 128), jnp.int32),
            pltpu.VMEM((1, GR * 128, D), jnp.float32),
            pltpu.VMEM((1, GR * 128, D), jnp.float32),
            pltpu.VMEM_SHARED((1, NROWS, D), jnp.float32),
            pltpu.SemaphoreType.DMA,
            pltpu.SemaphoreType.DMA,
        ],
    )
    def agg(u_hbm, src_hbm, dst_hbm, zero_hbm, out_hbm, srcv, dstv,
            bufa, bufb, acc, sema, semb):
        c = lax.axis_index("c")
        s = lax.axis_index("s")
        start = jnp.where(c == 0, s * (K0 // GR), (16 * K0 + s * K1) // GR)
        ngrp = jnp.where(c == 0, K0 // GR, K1 // GR)

        pltpu.sync_copy(src_hbm.at[pl.ds(start, KMAX // GR)], srcv)
        pltpu.sync_copy(dst_hbm.at[pl.ds(start, KMAX // GR)], dstv)

        pltpu.sync_copy(zero_hbm, acc.at[0, pl.ds(s * RPT, RPT)])

        plsc.subcore_barrier()

        pltpu.async_copy(u_hbm.at[srcv.at[0]], bufa, sema)
        pltpu.async_copy(u_hbm.at[srcv.at[1]], bufb, semb)

        def _pair(p, carry):
            g = p * 2
            pltpu.make_async_copy(u_hbm.at[srcv.at[g]], bufa, sema).wait()
            pltpu.sync_copy(bufa, acc.at[dstv.at[g]], add=True)

            @pl.when(g + 2 < ngrp)
            def _():
                pltpu.async_copy(u_hbm.at[srcv.at[g + 2]], bufa, sema)

            pltpu.make_async_copy(u_hbm.at[srcv.at[g + 1]], bufb, semb).wait()
            pltpu.sync_copy(bufb, acc.at[dstv.at[g + 1]], add=True)

            @pl.when(g + 3 < ngrp)
            def _():
                pltpu.async_copy(u_hbm.at[srcv.at[g + 3]], bufb, semb)

            return carry

        lax.fori_loop(0, ngrp // 2, _pair, 0)

        plsc.subcore_barrier()

        pltpu.sync_copy(acc.at[0, pl.ds(s * RPT, RPT)],
                        out_hbm.at[c, pl.ds(s * RPT, RPT)])

    return agg


@functools.cache
def _make_sc_deg():

    @functools.partial(
        pl.kernel,
        out_type=jax.ShapeDtypeStruct((2, NROWS, DD), jnp.float32),
        mesh=_get_mesh(),
        compiler_params=pltpu.CompilerParams(use_tc_tiling_on_sc=False),
        scratch_types=[
            pltpu.VMEM((KROWS, 128), jnp.int32),
            pltpu.VMEM((128, DD), jnp.float32),
            pltpu.VMEM_SHARED((NROWS, DD), jnp.float32),
        ],
    )
    def deg(dst_hbm, ones_hbm, zero_hbm, out_hbm, dstv, ones, acc):
        c = lax.axis_index("c")
        s = lax.axis_index("s")
        wid = s * 2 + c

        pltpu.sync_copy(dst_hbm.at[wid], dstv)
        pltpu.sync_copy(ones_hbm, ones)
        pltpu.sync_copy(zero_hbm, acc.at[pl.ds(s * RPT, RPT)])

        plsc.subcore_barrier()

        @pl.loop(0, KROWS)
        def _(j):
            pltpu.sync_copy(ones, acc.at[dstv.at[j]], add=True)

        plsc.subcore_barrier()

        pltpu.sync_copy(acc.at[pl.ds(s * RPT, RPT)],
                        out_hbm.at[c, pl.ds(s * RPT, RPT)])

    return deg


def _mm1_body(x_ref, w_ref, o_ref):
    o_ref[...] = jnp.dot(x_ref[...], w_ref[...],
                         preferred_element_type=jnp.float32)


def _tc_mm1(x, w1p):
    return pl.pallas_call(
        _mm1_body,
        out_shape=jax.ShapeDtypeStruct((N, D1), jnp.float32),
    )(x, w1p)


def _scale_body(degp_ref, xw_ref, u_ref, dinv_ref):
    deg = degp_ref[0, :N, 0:1] + degp_ref[1, :N, 0:1] + 1.0
    dinv = lax.rsqrt(deg)
    dinv_ref[...] = dinv
    u_ref[...] = xw_ref[...] * dinv


def _tc_scale(degp, xw):
    return pl.pallas_call(
        _scale_body,
        out_shape=(jax.ShapeDtypeStruct((N, D1), jnp.float32),
                   jax.ShapeDtypeStruct((N, 1), jnp.float32)),
    )(degp, xw)


def _layer_body(aggp_ref, u_ref, dinv_ref, b1_ref, w2_ref, u2_ref):
    dinv = dinv_ref[...]
    a = aggp_ref[0, :N, :] + aggp_ref[1, :N, :] + u_ref[...]
    h = jnp.maximum(a * dinv + b1_ref[...], 0.0)
    u2_ref[...] = jnp.dot(h, w2_ref[...],
                          preferred_element_type=jnp.float32) * dinv


def _tc_layer(aggp, u1, dinv, b1p, w2p):
    return pl.pallas_call(
        _layer_body,
        out_shape=jax.ShapeDtypeStruct((N, D2), jnp.float32),
    )(aggp, u1, dinv, b1p, w2p)


def _final_body(aggp_ref, u2_ref, dinv_ref, b2_ref, o_ref):
    a = aggp_ref[0, :N, :] + aggp_ref[1, :N, :] + u2_ref[...]
    o_ref[...] = (a * dinv_ref[...] + b2_ref[...])[:, :20]


def _tc_final(aggp, u2, dinv, b2p):
    return pl.pallas_call(
        _final_body,
        out_shape=jax.ShapeDtypeStruct((N, 20), jnp.float32),
    )(aggp, u2, dinv, b2p)


def kernel(x, edge_index, W1, b1, W2, b2):
    src = edge_index[0].astype(jnp.int32)
    dst = edge_index[1].astype(jnp.int32)
    npad = EPAD - E
    srcf = jnp.concatenate([src, jnp.zeros((npad,), jnp.int32)])
    dstf = jnp.concatenate(
        [dst, N + (jnp.arange(npad, dtype=jnp.int32) % (NROWS - N))])
    extra = (FLATROW - TOTROW) * 128
    srcp = jnp.concatenate(
        [srcf, jnp.zeros((extra,), jnp.int32)]
    ).reshape(FLATROW // GR, 1, GR * 128)
    dstp = jnp.concatenate(
        [dstf, jnp.full((extra,), N, jnp.int32)]
    ).reshape(FLATROW // GR, 1, GR * 128)
    dstp_deg = dstf.reshape(NTILES, KROWS, 128)

    w1p = jnp.pad(W1, ((0, 0), (0, D1 - 40)))
    b1p = jnp.pad(b1, (0, D1 - 40)).reshape(1, D1)
    w2p = jnp.pad(W2, ((0, D1 - 40), (0, D2 - 20)))
    b2p = jnp.pad(b2, (0, D2 - 20)).reshape(1, D2)

    ones_c = jnp.ones((128, DD), jnp.float32)
    zero_c = jnp.zeros((RPT, DD), jnp.float32)
    zero1 = jnp.zeros((RPT, D1), jnp.float32)
    zero2 = jnp.zeros((RPT, D2), jnp.float32)
    xw = _tc_mm1(x, w1p)
    degp = _make_sc_deg()(dstp_deg, ones_c, zero_c)
    u1, dinv = _tc_scale(degp, xw)
    agg1 = _make_sc_agg(D1)(u1.reshape(1, N, D1), srcp, dstp, zero1)
    u2 = _tc_layer(agg1, u1, dinv, b1p, w2p)
    agg2 = _make_sc_agg(D2)(u2.reshape(1, N, D2), srcp, dstp, zero2)
    return _tc_final(agg2, u2, dinv, b2p)


# speedup vs baseline: 1.3276x; 1.2903x over previous
import functools

import jax
import jax.numpy as jnp
from jax import lax
from jax.experimental import pallas as pl
from jax.experimental.pallas import tpu as pltpu
from jax.experimental.pallas import tpu_sc as plsc

N = 10000
E = 320000
NTILES = 32
KROWS = 80
EPAD = NTILES * KROWS * 128
K0 = 104
K1 = 2 * KROWS - K0
KMAX = max(K0, K1)
GR = 4
assert K0 % (2 * GR) == 0 and K1 % (2 * GR) == 0
TOTROW = 16 * (K0 + K1)
FLATROW = max(TOTROW, 16 * K0 + 15 * K1 + KMAX)
FLATROW += (-FLATROW) % GR
NROWS = 10240
RPT = NROWS // 16
D1 = 64
D2 = 32
DD = 8

@functools.cache
def _get_mesh():
    return plsc.VectorSubcoreMesh(core_axis_name="c", subcore_axis_name="s")


@functools.cache
def _make_sc_agg(D):
    @functools.partial(
        pl.kernel,
        out_type=jax.ShapeDtypeStruct((2, NROWS, D), jnp.bfloat16),
        mesh=_get_mesh(),
        compiler_params=pltpu.CompilerParams(use_tc_tiling_on_sc=False),
        scratch_types=[
            pltpu.VMEM((KMAX // GR, 1, GR * 128), jnp.int32),
            pltpu.VMEM((KMAX // GR, 1, GR * 128), jnp.int32),
            pltpu.VMEM((1, GR * 128, D), jnp.bfloat16),
            pltpu.VMEM((1, GR * 128, D), jnp.bfloat16),
            pltpu.VMEM_SHARED((1, NROWS, D), jnp.bfloat16),
            pltpu.SemaphoreType.DMA,
            pltpu.SemaphoreType.DMA,
        ],
    )
    def agg(u_hbm, src_hbm, dst_hbm, zero_hbm, out_hbm, srcv, dstv,
            bufa, bufb, acc, sema, semb):
        c = lax.axis_index("c")
        s = lax.axis_index("s")
        start = jnp.where(c == 0, s * (K0 // GR), (16 * K0 + s * K1) // GR)
        ngrp = jnp.where(c == 0, K0 // GR, K1 // GR)

        pltpu.sync_copy(src_hbm.at[pl.ds(start, KMAX // GR)], srcv)
        pltpu.sync_copy(dst_hbm.at[pl.ds(start, KMAX // GR)], dstv)

        pltpu.sync_copy(zero_hbm, acc.at[0, pl.ds(s * RPT, RPT)])

        plsc.subcore_barrier()

        pltpu.async_copy(u_hbm.at[srcv.at[0]], bufa, sema)
        pltpu.async_copy(u_hbm.at[srcv.at[1]], bufb, semb)

        def _pair(p, carry):
            g = p * 2
            pltpu.make_async_copy(u_hbm.at[srcv.at[g]], bufa, sema).wait()
            pltpu.sync_copy(bufa, acc.at[dstv.at[g]], add=True)

            @pl.when(g + 2 < ngrp)
            def _():
                pltpu.async_copy(u_hbm.at[srcv.at[g + 2]], bufa, sema)

            pltpu.make_async_copy(u_hbm.at[srcv.at[g + 1]], bufb, semb).wait()
            pltpu.sync_copy(bufb, acc.at[dstv.at[g + 1]], add=True)

            @pl.when(g + 3 < ngrp)
            def _():
                pltpu.async_copy(u_hbm.at[srcv.at[g + 3]], bufb, semb)

            return carry

        lax.fori_loop(0, ngrp // 2, _pair, 0)

        plsc.subcore_barrier()

        pltpu.sync_copy(acc.at[0, pl.ds(s * RPT, RPT)],
                        out_hbm.at[c, pl.ds(s * RPT, RPT)])

    return agg


@functools.cache
def _make_sc_deg():

    @functools.partial(
        pl.kernel,
        out_type=jax.ShapeDtypeStruct((2, NROWS, DD), jnp.float32),
        mesh=_get_mesh(),
        compiler_params=pltpu.CompilerParams(use_tc_tiling_on_sc=False),
        scratch_types=[
            pltpu.VMEM((KROWS, 128), jnp.int32),
            pltpu.VMEM((128, DD), jnp.float32),
            pltpu.VMEM_SHARED((NROWS, DD), jnp.float32),
        ],
    )
    def deg(dst_hbm, ones_hbm, zero_hbm, out_hbm, dstv, ones, acc):
        c = lax.axis_index("c")
        s = lax.axis_index("s")
        wid = s * 2 + c

        pltpu.sync_copy(dst_hbm.at[wid], dstv)
        pltpu.sync_copy(ones_hbm, ones)
        pltpu.sync_copy(zero_hbm, acc.at[pl.ds(s * RPT, RPT)])

        plsc.subcore_barrier()

        @pl.loop(0, KROWS)
        def _(j):
            pltpu.sync_copy(ones, acc.at[dstv.at[j]], add=True)

        plsc.subcore_barrier()

        pltpu.sync_copy(acc.at[pl.ds(s * RPT, RPT)],
                        out_hbm.at[c, pl.ds(s * RPT, RPT)])

    return deg


def _mm1_body(x_ref, w_ref, o_ref):
    o_ref[...] = jnp.dot(x_ref[...], w_ref[...],
                         preferred_element_type=jnp.float32)


def _tc_mm1(x, w1p):
    return pl.pallas_call(
        _mm1_body,
        out_shape=jax.ShapeDtypeStruct((N, D1), jnp.float32),
    )(x, w1p)


def _scale_body(degp_ref, xw_ref, u_ref, dinv_ref):
    deg = degp_ref[0, :N, 0:1] + degp_ref[1, :N, 0:1] + 1.0
    dinv = lax.rsqrt(deg)
    dinv_ref[...] = dinv
    u_ref[...] = (xw_ref[...] * dinv).astype(jnp.bfloat16)


def _tc_scale(degp, xw):
    return pl.pallas_call(
        _scale_body,
        out_shape=(jax.ShapeDtypeStruct((N, D1), jnp.bfloat16),
                   jax.ShapeDtypeStruct((N, 1), jnp.float32)),
    )(degp, xw)


def _layer_body(aggp_ref, xw_ref, dinv_ref, b1_ref, w2_ref, u2_ref, u2f_ref):
    dinv = dinv_ref[...]
    a = (aggp_ref[0, :N, :].astype(jnp.float32)
         + aggp_ref[1, :N, :].astype(jnp.float32)
         + xw_ref[...] * dinv)
    h = jnp.maximum(a * dinv + b1_ref[...], 0.0)
    u2 = jnp.dot(h, w2_ref[...], preferred_element_type=jnp.float32) * dinv
    u2f_ref[...] = u2
    u2_ref[...] = u2.astype(jnp.bfloat16)


def _tc_layer(aggp, xw, dinv, b1p, w2p):
    return pl.pallas_call(
        _layer_body,
        out_shape=(jax.ShapeDtypeStruct((N, D2), jnp.bfloat16),
                   jax.ShapeDtypeStruct((N, D2), jnp.float32)),
    )(aggp, xw, dinv, b1p, w2p)


def _final_body(aggp_ref, u2_ref, dinv_ref, b2_ref, o_ref):
    a = (aggp_ref[0, :N, :].astype(jnp.float32)
         + aggp_ref[1, :N, :].astype(jnp.float32) + u2_ref[...])
    o_ref[...] = (a * dinv_ref[...] + b2_ref[...])[:, :20]


def _tc_final(aggp, u2, dinv, b2p):
    return pl.pallas_call(
        _final_body,
        out_shape=jax.ShapeDtypeStruct((N, 20), jnp.float32),
    )(aggp, u2, dinv, b2p)


def kernel(x, edge_index, W1, b1, W2, b2):
    src = edge_index[0].astype(jnp.int32)
    dst = edge_index[1].astype(jnp.int32)
    npad = EPAD - E
    srcf = jnp.concatenate([src, jnp.zeros((npad,), jnp.int32)])
    dstf = jnp.concatenate(
        [dst, N + (jnp.arange(npad, dtype=jnp.int32) % (NROWS - N))])
    extra = (FLATROW - TOTROW) * 128
    srcp = jnp.concatenate(
        [srcf, jnp.zeros((extra,), jnp.int32)]
    ).reshape(FLATROW // GR, 1, GR * 128)
    dstp = jnp.concatenate(
        [dstf, jnp.full((extra,), N, jnp.int32)]
    ).reshape(FLATROW // GR, 1, GR * 128)
    dstp_deg = dstf.reshape(NTILES, KROWS, 128)

    w1p = jnp.pad(W1, ((0, 0), (0, D1 - 40)))
    b1p = jnp.pad(b1, (0, D1 - 40)).reshape(1, D1)
    w2p = jnp.pad(W2, ((0, D1 - 40), (0, D2 - 20)))
    b2p = jnp.pad(b2, (0, D2 - 20)).reshape(1, D2)

    ones_c = jnp.ones((128, DD), jnp.float32)
    zero_c = jnp.zeros((RPT, DD), jnp.float32)
    zero1 = jnp.zeros((RPT, D1), jnp.bfloat16)
    zero2 = jnp.zeros((RPT, D2), jnp.bfloat16)
    xw = _tc_mm1(x, w1p)
    degp = _make_sc_deg()(dstp_deg, ones_c, zero_c)
    u1, dinv = _tc_scale(degp, xw)
    agg1 = _make_sc_agg(D1)(u1.reshape(1, N, D1), srcp, dstp, zero1)
    u2b, u2f = _tc_layer(agg1, xw, dinv, b1p, w2p)
    agg2 = _make_sc_agg(D2)(u2b.reshape(1, N, D2), srcp, dstp, zero2)
    return _tc_final(agg2, u2f, dinv, b2p)
